# baseline (speedup 1.0000x reference)
import functools

import jax
import jax.numpy as jnp
from jax import lax
from jax.experimental import pallas as pl
from jax.experimental.pallas import tpu as pltpu
from jax.experimental.pallas import tpu_sc as plsc

N = 10000
E = 320000
D = 128
D2 = D // 2

NC = 2
NS = 16
NW = NC * NS

NPAD = 10240
EW = E // NW
CH = 80
NCH = EW // CH

_SC_PARAMS = pltpu.CompilerParams(
    needs_layout_passes=False, use_tc_tiling_on_sc=False
)
_MESH = plsc.VectorSubcoreMesh(core_axis_name="c", subcore_axis_name="s")


def _project_body(h_ref, ws_ref, bs_ref, wt_ref, bt_ref, zs_ref, zt_ref):
    h = h_ref[...]
    dims = (((1,), (1,)), ((), ()))
    zs_ref[...] = (
        lax.dot_general(h, ws_ref[...], dims, preferred_element_type=jnp.float32)
        + bs_ref[...]
    ).astype(jnp.bfloat16)
    zt_ref[...] = (
        lax.dot_general(h, wt_ref[...], dims, preferred_element_type=jnp.float32)
        + bt_ref[...]
    ).astype(jnp.bfloat16)


def _project(hidden, Ws, bs2, Wt, bt2):
    rb = 1000
    grid = (N // rb,)
    return pl.pallas_call(
        _project_body,
        grid=grid,
        in_specs=[
            pl.BlockSpec((rb, D), lambda i: (i, 0)),
            pl.BlockSpec((D, D), lambda i: (0, 0)),
            pl.BlockSpec((1, D), lambda i: (0, 0)),
            pl.BlockSpec((D, D), lambda i: (0, 0)),
            pl.BlockSpec((1, D), lambda i: (0, 0)),
        ],
        out_specs=[
            pl.BlockSpec((rb, D), lambda i: (i, 0)),
            pl.BlockSpec((rb, D), lambda i: (i, 0)),
        ],
        out_shape=[
            jax.ShapeDtypeStruct((N, D), jnp.bfloat16),
            jax.ShapeDtypeStruct((N, D), jnp.bfloat16),
        ],
    )(hidden, Ws, bs2, Wt, bt2)


@functools.partial(
    pl.kernel,
    compiler_params=_SC_PARAMS,
    out_type=(
        jax.ShapeDtypeStruct((NW, EW), jnp.float32),
        jax.ShapeDtypeStruct((NW, NPAD), jnp.float32),
    ),
    mesh=_MESH,
    scratch_types=[
        pltpu.VMEM((EW,), jnp.int32),
        pltpu.VMEM((EW,), jnp.int32),
        pltpu.VMEM((CH, D2), jnp.int32),
        pltpu.VMEM((CH, D2), jnp.int32),
        pltpu.VMEM((CH, D2), jnp.int32),
        pltpu.VMEM((CH, D2), jnp.int32),
        pltpu.VMEM((CH, D2), jnp.int32),
        pltpu.VMEM((CH, D2), jnp.int32),
        pltpu.VMEM((CH, D2), jnp.int32),
        pltpu.VMEM((CH, D2), jnp.int32),
        pltpu.VMEM((EW,), jnp.float32),
        pltpu.VMEM((NPAD,), jnp.float32),
        pltpu.SemaphoreType.DMA,
        pltpu.SemaphoreType.DMA,
        pltpu.SemaphoreType.DMA,
        pltpu.SemaphoreType.DMA,
    ],
)
def _edge_scores(zs_hbm, zt_hbm, edge_hbm, z_out, p_out,
                 srcv, dstv, rs0, rt0, rs1, rt1, rs2, rt2, rs3, rt3,
                 zv, dn, sem0, sem1, sem2, sem3):
    wid = lax.axis_index("s") * NC + lax.axis_index("c")
    pltpu.sync_copy(edge_hbm.at[0, pl.ds(wid * EW, EW)], srcv)
    pltpu.sync_copy(edge_hbm.at[1, pl.ds(wid * EW, EW)], dstv)

    def zero_body(i, _):
        dn[pl.ds(i * 16, 16)] = jnp.zeros((16,), jnp.float32)
        return 0

    lax.fori_loop(0, NPAD // 16, zero_body, 0)

    def fire(c, rs, rt, sem):
        pltpu.async_copy(zs_hbm.at[srcv.at[pl.ds(c * CH, CH)]], rs, sem)
        pltpu.async_copy(zt_hbm.at[dstv.at[pl.ds(c * CH, CH)]], rt, sem)

    def wait2(rs, rt, sem):
        pltpu.make_async_copy(zs_hbm.at[pl.ds(0, CH)], rs, sem).wait()
        pltpu.make_async_copy(zt_hbm.at[pl.ds(0, CH)], rt, sem).wait()

    def compute(c, rs, rt):
        lanes = lax.iota(jnp.int32, 16)

        def group(g, _):
            eidx = lanes + g * 16
            acc = jnp.zeros((16,), jnp.float32)
            for t in range(D2):
                dv = (lanes + t) & (D2 - 1)
                wa = plsc.load_gather(rs, [eidx, dv])
                wb = plsc.load_gather(rt, [eidx, dv])
                a0, a1 = plsc.unpack(
                    plsc.bitcast(wa, jnp.bfloat16), format=plsc.PackFormat.INTERLEAVED
                )
                b0, b1 = plsc.unpack(
                    plsc.bitcast(wb, jnp.bfloat16), format=plsc.PackFormat.INTERLEAVED
                )
                acc = acc + a0 * b0 + a1 * b1
            off = c * CH + g * 16
            zv[pl.ds(off, 16)] = acc
            keys = srcv[pl.ds(off, 16)]
            plsc.addupdate_scatter(dn, [keys], jnp.exp(acc))
            return 0

        lax.fori_loop(0, CH // 16, group, 0)

    bufs = ((rs0, rt0, sem0), (rs1, rt1, sem1), (rs2, rt2, sem2), (rs3, rt3, sem3))
    for k in range(4):
        fire(k, *bufs[k])

    def loop(j4, _):
        for k in range(4):
            c = 4 * j4 + k
            rs, rt, sem = bufs[k]
            wait2(rs, rt, sem)
            compute(c, rs, rt)

            @pl.when(c + 4 < NCH)
            def _():
                fire(c + 4, rs, rt, sem)

        return 0

    lax.fori_loop(0, NCH // 4, loop, 0)
    wait2(rs0, rt0, sem0)
    compute(NCH - 1, rs0, rt0)

    pltpu.sync_copy(zv, z_out.at[wid])
    pltpu.sync_copy(dn, p_out.at[wid])


def _log_combine_body(p_ref, c_ref):
    c_ref[...] = jnp.log(jnp.sum(p_ref[...], axis=0, keepdims=True))


def _log_combine(partials):
    return pl.pallas_call(
        _log_combine_body,
        out_shape=jax.ShapeDtypeStruct((1, NPAD), jnp.float32),
    )(partials)


@functools.partial(
    pl.kernel,
    compiler_params=_SC_PARAMS,
    out_type=jax.ShapeDtypeStruct((NW, EW), jnp.float32),
    mesh=_MESH,
    scratch_types=[
        pltpu.VMEM((NPAD,), jnp.float32),
        pltpu.VMEM((EW,), jnp.float32),
        pltpu.VMEM((EW,), jnp.int32),
        pltpu.VMEM((EW,), jnp.float32),
    ],
)
def _edge_output(z_hbm, edge_hbm, c_hbm, out_hbm, cv, zv, srcv, outv):
    wid = lax.axis_index("s") * NC + lax.axis_index("c")
    pltpu.sync_copy(c_hbm, cv)
    pltpu.sync_copy(z_hbm.at[wid], zv)
    pltpu.sync_copy(edge_hbm.at[0, pl.ds(wid * EW, EW)], srcv)

    def group(g, _):
        off = g * 16
        keys = srcv[pl.ds(off, 16)]
        cg = plsc.load_gather(cv, [keys])
        outv[pl.ds(off, 16)] = zv[pl.ds(off, 16)] - cg
        return 0

    lax.fori_loop(0, EW // 16, group, 0)
    pltpu.sync_copy(outv, out_hbm.at[wid])


def kernel(hidden, edge_index, Ws, bs, Wt, bt):
    zs, zt = _project(hidden, Ws, bs.reshape(1, D), Wt, bt.reshape(1, D))
    zs = lax.bitcast_convert_type(zs.reshape(N, D2, 2), jnp.int32)
    zt = lax.bitcast_convert_type(zt.reshape(N, D2, 2), jnp.int32)

    z, partials = _edge_scores(zs, zt, edge_index)
    c = _log_combine(partials).reshape(NPAD)
    out = _edge_output(z, edge_index, c)
    return out.reshape(E)


# speedup vs baseline: 1.5378x; 1.5378x over previous
import functools

import jax
import jax.numpy as jnp
from jax import lax
from jax.experimental import pallas as pl
from jax.experimental.pallas import tpu as pltpu
from jax.experimental.pallas import tpu_sc as plsc

N = 10000
E = 320000
D = 128
D2 = D // 2

NC = 2
NS = 16
NW = NC * NS

NPAD = 10240
EW = E // NW
CH = 80
NCH = EW // CH

_SC_PARAMS = pltpu.CompilerParams(
    needs_layout_passes=False, use_tc_tiling_on_sc=False
)
_MESH = plsc.VectorSubcoreMesh(core_axis_name="c", subcore_axis_name="s")


def _project_body(h_ref, ws_ref, bs_ref, wt_ref, bt_ref, zs_ref, zt_ref):
    h = h_ref[...]
    dims = (((1,), (1,)), ((), ()))
    zs_ref[...] = (
        lax.dot_general(h, ws_ref[...], dims, preferred_element_type=jnp.float32)
        + bs_ref[...]
    ).astype(jnp.bfloat16)
    zt_ref[...] = (
        lax.dot_general(h, wt_ref[...], dims, preferred_element_type=jnp.float32)
        + bt_ref[...]
    ).astype(jnp.bfloat16)


def _project(hidden, Ws, bs2, Wt, bt2):
    rb = 1000
    grid = (N // rb,)
    return pl.pallas_call(
        _project_body,
        grid=grid,
        in_specs=[
            pl.BlockSpec((rb, D), lambda i: (i, 0)),
            pl.BlockSpec((D, D), lambda i: (0, 0)),
            pl.BlockSpec((1, D), lambda i: (0, 0)),
            pl.BlockSpec((D, D), lambda i: (0, 0)),
            pl.BlockSpec((1, D), lambda i: (0, 0)),
        ],
        out_specs=[
            pl.BlockSpec((rb, D), lambda i: (i, 0)),
            pl.BlockSpec((rb, D), lambda i: (i, 0)),
        ],
        out_shape=[
            jax.ShapeDtypeStruct((N, D), jnp.bfloat16),
            jax.ShapeDtypeStruct((N, D), jnp.bfloat16),
        ],
    )(hidden, Ws, bs2, Wt, bt2)


@functools.partial(
    pl.kernel,
    compiler_params=_SC_PARAMS,
    out_type=(
        jax.ShapeDtypeStruct((NW, EW), jnp.float32),
        jax.ShapeDtypeStruct((NW, NPAD), jnp.float32),
    ),
    mesh=_MESH,
    scratch_types=[
        pltpu.VMEM((EW,), jnp.int32),
        pltpu.VMEM((EW,), jnp.int32),
        pltpu.VMEM((CH, D2), jnp.int32),
        pltpu.VMEM((CH, D2), jnp.int32),
        pltpu.VMEM((CH, D2), jnp.int32),
        pltpu.VMEM((CH, D2), jnp.int32),
        pltpu.VMEM((EW,), jnp.float32),
        pltpu.VMEM((NPAD,), jnp.float32),
        pltpu.SemaphoreType.DMA,
        pltpu.SemaphoreType.DMA,
    ],
)
def _edge_scores(zs_hbm, zt_hbm, edge_hbm, z_out, p_out,
                 srcv, dstv, rs0, rt0, rs1, rt1, zv, dn, sem0, sem1):
    wid = lax.axis_index("s") * NC + lax.axis_index("c")
    pltpu.sync_copy(edge_hbm.at[0, pl.ds(wid * EW, EW)], srcv)
    pltpu.sync_copy(edge_hbm.at[1, pl.ds(wid * EW, EW)], dstv)

    def zero_body(i, _):
        dn[pl.ds(i * 16, 16)] = jnp.zeros((16,), jnp.float32)
        return 0

    lax.fori_loop(0, NPAD // 16, zero_body, 0)

    def fire(c, rs, rt, sem):
        pltpu.async_copy(zs_hbm.at[srcv.at[pl.ds(c * CH, CH)]], rs, sem)
        pltpu.async_copy(zt_hbm.at[dstv.at[pl.ds(c * CH, CH)]], rt, sem)

    def wait2(rs, rt, sem):
        pltpu.make_async_copy(zs_hbm.at[pl.ds(0, CH)], rs, sem).wait()
        pltpu.make_async_copy(zt_hbm.at[pl.ds(0, CH)], rt, sem).wait()

    def compute(c, rs, rt):
        lanes = lax.iota(jnp.int32, 16)

        def group(g, _):
            eidx = lanes + g * 16
            accs = [jnp.zeros((16,), jnp.float32) for _ in range(8)]
            for t in range(D2):
                dv = (lanes + t) & (D2 - 1)
                wa = plsc.load_gather(rs, [eidx, dv])
                wb = plsc.load_gather(rt, [eidx, dv])
                a0, a1 = plsc.unpack(
                    plsc.bitcast(wa, jnp.bfloat16), format=plsc.PackFormat.INTERLEAVED
                )
                b0, b1 = plsc.unpack(
                    plsc.bitcast(wb, jnp.bfloat16), format=plsc.PackFormat.INTERLEAVED
                )
                k = (t & 3) * 2
                accs[k] = accs[k] + a0 * b0
                accs[k + 1] = accs[k + 1] + a1 * b1
            acc = (
                (accs[0] + accs[1])
                + (accs[2] + accs[3])
                + ((accs[4] + accs[5]) + (accs[6] + accs[7]))
            )
            off = c * CH + g * 16
            zv[pl.ds(off, 16)] = acc
            keys = srcv[pl.ds(off, 16)]
            plsc.addupdate_scatter(dn, [keys], jnp.exp(acc))
            return 0

        lax.fori_loop(0, CH // 16, group, 0)

    fire(0, rs0, rt0, sem0)

    def loop(j2, _):
        c0 = 2 * j2
        c1 = c0 + 1
        fire(c1, rs1, rt1, sem1)
        wait2(rs0, rt0, sem0)
        compute(c0, rs0, rt0)

        @pl.when(c1 + 1 < NCH)
        def _():
            fire(c1 + 1, rs0, rt0, sem0)

        wait2(rs1, rt1, sem1)
        compute(c1, rs1, rt1)
        return 0

    lax.fori_loop(0, NCH // 2, loop, 0)
    wait2(rs0, rt0, sem0)
    compute(NCH - 1, rs0, rt0)

    pltpu.sync_copy(zv, z_out.at[wid])
    pltpu.sync_copy(dn, p_out.at[wid])


def _log_combine_body(p_ref, c_ref):
    c_ref[...] = jnp.log(jnp.sum(p_ref[...], axis=0, keepdims=True))


def _log_combine(partials):
    return pl.pallas_call(
        _log_combine_body,
        out_shape=jax.ShapeDtypeStruct((1, NPAD), jnp.float32),
    )(partials)


@functools.partial(
    pl.kernel,
    compiler_params=_SC_PARAMS,
    out_type=jax.ShapeDtypeStruct((NW, EW), jnp.float32),
    mesh=_MESH,
    scratch_types=[
        pltpu.VMEM((NPAD,), jnp.float32),
        pltpu.VMEM((EW,), jnp.float32),
        pltpu.VMEM((EW,), jnp.int32),
        pltpu.VMEM((EW,), jnp.float32),
    ],
)
def _edge_output(z_hbm, edge_hbm, c_hbm, out_hbm, cv, zv, srcv, outv):
    wid = lax.axis_index("s") * NC + lax.axis_index("c")
    pltpu.sync_copy(c_hbm, cv)
    pltpu.sync_copy(z_hbm.at[wid], zv)
    pltpu.sync_copy(edge_hbm.at[0, pl.ds(wid * EW, EW)], srcv)

    def group(g, _):
        off = g * 16
        keys = srcv[pl.ds(off, 16)]
        cg = plsc.load_gather(cv, [keys])
        outv[pl.ds(off, 16)] = zv[pl.ds(off, 16)] - cg
        return 0

    lax.fori_loop(0, EW // 16, group, 0)
    pltpu.sync_copy(outv, out_hbm.at[wid])


def kernel(hidden, edge_index, Ws, bs, Wt, bt):
    zs, zt = _project(hidden, Ws, bs.reshape(1, D), Wt, bt.reshape(1, D))
    zs = lax.bitcast_convert_type(zs.reshape(N, D2, 2), jnp.int32)
    zt = lax.bitcast_convert_type(zt.reshape(N, D2, 2), jnp.int32)

    z, partials = _edge_scores(zs, zt, edge_index)
    c = _log_combine(partials).reshape(NPAD)
    out = _edge_output(z, edge_index, c)
    return out.reshape(E)


# speedup vs baseline: 1.5934x; 1.0361x over previous
import functools

import jax
import jax.numpy as jnp
from jax import lax
from jax.experimental import pallas as pl
from jax.experimental.pallas import tpu as pltpu
from jax.experimental.pallas import tpu_sc as plsc

N = 10000
E = 320000
D = 128
D2 = D // 2

NC = 2
NS = 16
NW = NC * NS

NPAD = 10240
EW = E // NW
CH = 80
NCH = EW // CH

_SC_PARAMS = pltpu.CompilerParams(
    needs_layout_passes=False, use_tc_tiling_on_sc=False
)
_MESH = plsc.VectorSubcoreMesh(core_axis_name="c", subcore_axis_name="s")


def _project_body(h_ref, ws_ref, bs_ref, wt_ref, bt_ref, zs_ref, zt_ref):
    h = h_ref[...]
    dims = (((1,), (1,)), ((), ()))
    zs_ref[...] = (
        lax.dot_general(h, ws_ref[...], dims, preferred_element_type=jnp.float32)
        + bs_ref[...]
    ).astype(jnp.bfloat16)
    zt_ref[...] = (
        lax.dot_general(h, wt_ref[...], dims, preferred_element_type=jnp.float32)
        + bt_ref[...]
    ).astype(jnp.bfloat16)


def _project(hidden, Ws, bs2, Wt, bt2):
    rb = 1000
    grid = (N // rb,)
    return pl.pallas_call(
        _project_body,
        grid=grid,
        in_specs=[
            pl.BlockSpec((rb, D), lambda i: (i, 0)),
            pl.BlockSpec((D, D), lambda i: (0, 0)),
            pl.BlockSpec((1, D), lambda i: (0, 0)),
            pl.BlockSpec((D, D), lambda i: (0, 0)),
            pl.BlockSpec((1, D), lambda i: (0, 0)),
        ],
        out_specs=[
            pl.BlockSpec((rb, D), lambda i: (i, 0)),
            pl.BlockSpec((rb, D), lambda i: (i, 0)),
        ],
        out_shape=[
            jax.ShapeDtypeStruct((N, D), jnp.bfloat16),
            jax.ShapeDtypeStruct((N, D), jnp.bfloat16),
        ],
    )(hidden, Ws, bs2, Wt, bt2)


@functools.partial(
    pl.kernel,
    compiler_params=_SC_PARAMS,
    out_type=(
        jax.ShapeDtypeStruct((NW, EW), jnp.float32),
        jax.ShapeDtypeStruct((NW, NPAD), jnp.float32),
    ),
    mesh=_MESH,
    scratch_types=[
        pltpu.VMEM((EW,), jnp.int32),
        pltpu.VMEM((EW,), jnp.int32),
        pltpu.VMEM((CH, D2), jnp.int32),
        pltpu.VMEM((CH, D2), jnp.int32),
        pltpu.VMEM((CH, D2), jnp.int32),
        pltpu.VMEM((CH, D2), jnp.int32),
        pltpu.VMEM((EW,), jnp.float32),
        pltpu.VMEM((NPAD,), jnp.float32),
        pltpu.SemaphoreType.DMA,
        pltpu.SemaphoreType.DMA,
    ],
)
def _edge_scores(zs_hbm, zt_hbm, edge_hbm, z_out, p_out,
                 srcv, dstv, rs0, rt0, rs1, rt1, zv, dn, sem0, sem1):
    wid = lax.axis_index("s") * NC + lax.axis_index("c")
    pltpu.sync_copy(edge_hbm.at[0, pl.ds(wid * EW, EW)], srcv)
    pltpu.sync_copy(edge_hbm.at[1, pl.ds(wid * EW, EW)], dstv)

    def zero_body(i, _):
        dn[pl.ds(i * 16, 16)] = jnp.zeros((16,), jnp.float32)
        return 0

    lax.fori_loop(0, NPAD // 16, zero_body, 0)

    def fire(c, rs, rt, sem):
        pltpu.async_copy(zs_hbm.at[srcv.at[pl.ds(c * CH, CH)]], rs, sem)
        pltpu.async_copy(zt_hbm.at[dstv.at[pl.ds(c * CH, CH)]], rt, sem)

    def wait2(rs, rt, sem):
        pltpu.make_async_copy(zs_hbm.at[pl.ds(0, CH)], rs, sem).wait()
        pltpu.make_async_copy(zt_hbm.at[pl.ds(0, CH)], rt, sem).wait()

    def compute(c, rs, rt):
        lanes = lax.iota(jnp.int32, 16)

        def group(g, _):
            eidx = lanes + g * 16
            accs = [jnp.zeros((16,), jnp.float32) for _ in range(8)]
            for t in range(D2):
                dv = (lanes + t) & (D2 - 1)
                wa = plsc.load_gather(rs, [eidx, dv])
                wb = plsc.load_gather(rt, [eidx, dv])
                p = plsc.bitcast(wa, jnp.bfloat16) * plsc.bitcast(wb, jnp.bfloat16)
                p0, p1 = plsc.unpack(p, format=plsc.PackFormat.INTERLEAVED)
                k = (t & 3) * 2
                accs[k] = accs[k] + p0
                accs[k + 1] = accs[k + 1] + p1
            acc = (
                (accs[0] + accs[1])
                + (accs[2] + accs[3])
                + ((accs[4] + accs[5]) + (accs[6] + accs[7]))
            )
            off = c * CH + g * 16
            zv[pl.ds(off, 16)] = acc
            keys = srcv[pl.ds(off, 16)]
            plsc.addupdate_scatter(dn, [keys], jnp.exp(acc))
            return 0

        lax.fori_loop(0, CH // 16, group, 0)

    fire(0, rs0, rt0, sem0)

    def loop(j2, _):
        c0 = 2 * j2
        c1 = c0 + 1
        fire(c1, rs1, rt1, sem1)
        wait2(rs0, rt0, sem0)
        compute(c0, rs0, rt0)

        @pl.when(c1 + 1 < NCH)
        def _():
            fire(c1 + 1, rs0, rt0, sem0)

        wait2(rs1, rt1, sem1)
        compute(c1, rs1, rt1)
        return 0

    lax.fori_loop(0, NCH // 2, loop, 0)
    wait2(rs0, rt0, sem0)
    compute(NCH - 1, rs0, rt0)

    pltpu.sync_copy(zv, z_out.at[wid])
    pltpu.sync_copy(dn, p_out.at[wid])


def _log_combine_body(p_ref, c_ref):
    c_ref[...] = jnp.log(jnp.sum(p_ref[...], axis=0, keepdims=True))


def _log_combine(partials):
    return pl.pallas_call(
        _log_combine_body,
        out_shape=jax.ShapeDtypeStruct((1, NPAD), jnp.float32),
    )(partials)


@functools.partial(
    pl.kernel,
    compiler_params=_SC_PARAMS,
    out_type=jax.ShapeDtypeStruct((NW, EW), jnp.float32),
    mesh=_MESH,
    scratch_types=[
        pltpu.VMEM((NPAD,), jnp.float32),
        pltpu.VMEM((EW,), jnp.float32),
        pltpu.VMEM((EW,), jnp.int32),
        pltpu.VMEM((EW,), jnp.float32),
    ],
)
def _edge_output(z_hbm, edge_hbm, c_hbm, out_hbm, cv, zv, srcv, outv):
    wid = lax.axis_index("s") * NC + lax.axis_index("c")
    pltpu.sync_copy(c_hbm, cv)
    pltpu.sync_copy(z_hbm.at[wid], zv)
    pltpu.sync_copy(edge_hbm.at[0, pl.ds(wid * EW, EW)], srcv)

    def group(g, _):
        off = g * 16
        keys = srcv[pl.ds(off, 16)]
        cg = plsc.load_gather(cv, [keys])
        outv[pl.ds(off, 16)] = zv[pl.ds(off, 16)] - cg
        return 0

    lax.fori_loop(0, EW // 16, group, 0)
    pltpu.sync_copy(outv, out_hbm.at[wid])


def kernel(hidden, edge_index, Ws, bs, Wt, bt):
    zs, zt = _project(hidden, Ws, bs.reshape(1, D), Wt, bt.reshape(1, D))
    zs = lax.bitcast_convert_type(zs.reshape(N, D2, 2), jnp.int32)
    zt = lax.bitcast_convert_type(zt.reshape(N, D2, 2), jnp.int32)

    z, partials = _edge_scores(zs, zt, edge_index)
    c = _log_combine(partials).reshape(NPAD)
    out = _edge_output(z, edge_index, c)
    return out.reshape(E)
